# Initial kernel scaffold; baseline (speedup 1.0000x reference)
import jax
import jax.numpy as jnp
from jax.experimental import pallas as pl


def kernel(x, edge_index, W1, b1, W2, b2):
    raise NotImplementedError("write your pallas kernel here")


# speedup vs baseline: 2.8778x; 2.8778x over previous
import functools

import jax
import jax.numpy as jnp
from jax import lax
from jax.experimental import pallas as pl
from jax.experimental.pallas import tpu as pltpu
from jax.experimental.pallas import tpu_sc as plsc

N = 10000
D = 128
DW = 144
NROWS = 10240
E = 320000
EPAD = 327680
EROWS = 2560
SENT = N
NWORK = 32
RPW = EROWS // NWORK
CH = 8
TPT = NROWS // 16

BLK = 512


def _mm_pad_body(x_ref, w_ref, o_ref):
    t = jnp.dot(x_ref[...], w_ref[...], preferred_element_type=jnp.float32)
    tail = jnp.where(
        lax.broadcasted_iota(jnp.int32, (BLK, 16), 1) == 0, 1.0, 0.0
    ).astype(jnp.float32)
    o_ref[...] = jnp.concatenate([t, tail], axis=1)


def _mm_pad(xp, W):
    return pl.pallas_call(
        _mm_pad_body,
        grid=(NROWS // BLK,),
        in_specs=[
            pl.BlockSpec((BLK, D), lambda i: (i, 0)),
            pl.BlockSpec((D, D), lambda i: (0, 0)),
        ],
        out_specs=pl.BlockSpec((BLK, DW), lambda i: (i, 0)),
        out_shape=jax.ShapeDtypeStruct((NROWS, DW), jnp.float32),
    )(xp, W)


def _mid_body(agg_ref, t_ref, b_ref, w_ref, h1_ref, t2_ref):
    s = agg_ref[0] + agg_ref[1]
    deg = s[:, 128:129]
    h = (s[:, :128] + t_ref[:, :128]) / (deg + 1.0) + b_ref[...]
    h1 = jnp.maximum(h, 0.0)
    h1_ref[...] = h1
    t2 = jnp.dot(h1, w_ref[...], preferred_element_type=jnp.float32)
    tail = jnp.where(
        lax.broadcasted_iota(jnp.int32, (BLK, 16), 1) == 0, 1.0, 0.0
    ).astype(jnp.float32)
    t2_ref[...] = jnp.concatenate([t2, tail], axis=1)


def _mid(agg, t1p, b1, W2):
    return pl.pallas_call(
        _mid_body,
        grid=(NROWS // BLK,),
        in_specs=[
            pl.BlockSpec((2, BLK, DW), lambda i: (0, i, 0)),
            pl.BlockSpec((BLK, DW), lambda i: (i, 0)),
            pl.BlockSpec((1, D), lambda i: (0, 0)),
            pl.BlockSpec((D, D), lambda i: (0, 0)),
        ],
        out_specs=[
            pl.BlockSpec((BLK, D), lambda i: (i, 0)),
            pl.BlockSpec((BLK, DW), lambda i: (i, 0)),
        ],
        out_shape=[
            jax.ShapeDtypeStruct((NROWS, D), jnp.float32),
            jax.ShapeDtypeStruct((NROWS, DW), jnp.float32),
        ],
    )(agg, t1p, b1, W2)


def _fin_body(agg_ref, t_ref, b_ref, h2_ref):
    s = agg_ref[0] + agg_ref[1]
    deg = s[:, 128:129]
    h2_ref[...] = (s[:, :128] + t_ref[:, :128]) / (deg + 1.0) + b_ref[...]


def _fin(agg, t2p, b2):
    return pl.pallas_call(
        _fin_body,
        grid=(NROWS // BLK,),
        in_specs=[
            pl.BlockSpec((2, BLK, DW), lambda i: (0, i, 0)),
            pl.BlockSpec((BLK, DW), lambda i: (i, 0)),
            pl.BlockSpec((1, D), lambda i: (0, 0)),
        ],
        out_specs=pl.BlockSpec((BLK, D), lambda i: (i, 0)),
        out_shape=jax.ShapeDtypeStruct((NROWS, D), jnp.float32),
    )(agg, t2p, b2)


def _sc_agg_body(t_hbm, src_hbm, dst_hbm, out_hbm, src_v, dst_v, rows_v, zbuf_v,
                 acc_sh):
    cid = lax.axis_index("c")
    sid = lax.axis_index("s")
    wid = cid * 16 + sid

    z16 = jnp.zeros((16,), jnp.float32)
    for r in range(16):
        for c in range(DW // 16):
            zbuf_v[r, 16 * c:16 * (c + 1)] = z16

    def zacc(i, carry):
        pltpu.sync_copy(zbuf_v, acc_sh.at[pl.ds(sid * TPT + i * 16, 16)])
        return carry

    lax.fori_loop(0, TPT // 16, zacc, 0)
    plsc.subcore_barrier()

    wbase = wid * RPW

    def chunk(i, carry):
        r0 = wbase + i * CH
        pltpu.sync_copy(src_hbm.at[pl.ds(r0, CH)], src_v)
        pltpu.sync_copy(dst_hbm.at[pl.ds(r0, CH)], dst_v)
        for j in range(CH):
            pltpu.sync_copy(t_hbm.at[src_v.at[j]], rows_v)
            pltpu.sync_copy(rows_v, acc_sh.at[dst_v.at[j]], add=True)
        return carry

    lax.fori_loop(0, RPW // CH, chunk, 0)
    plsc.subcore_barrier()

    pltpu.sync_copy(acc_sh.at[pl.ds(sid * TPT, TPT)],
                    out_hbm.at[cid, pl.ds(sid * TPT, TPT)])


@functools.partial(
    pl.kernel,
    mesh=plsc.VectorSubcoreMesh(core_axis_name="c", subcore_axis_name="s"),
    compiler_params=pltpu.CompilerParams(use_tc_tiling_on_sc=False),
    out_type=jax.ShapeDtypeStruct((2, NROWS, DW), jnp.float32),
    scratch_types=[
        pltpu.VMEM((CH, 128), jnp.int32),
        pltpu.VMEM((CH, 128), jnp.int32),
        pltpu.VMEM((128, DW), jnp.float32),
        pltpu.VMEM((16, DW), jnp.float32),
        pltpu.VMEM_SHARED((NROWS, DW), jnp.float32),
    ],
)
def _sc_agg(t_hbm, src_hbm, dst_hbm, out_hbm, src_v, dst_v, rows_v, zbuf_v,
            acc_sh):
    _sc_agg_body(t_hbm, src_hbm, dst_hbm, out_hbm, src_v, dst_v, rows_v,
                 zbuf_v, acc_sh)


def kernel(x, edge_index, W1, b1, W2, b2):
    src = edge_index[0]
    dst = edge_index[1]
    pad = jnp.full((EPAD - E,), SENT, jnp.int32)
    src2d = jnp.concatenate([src, pad]).reshape(EROWS, 128)
    dst2d = jnp.concatenate([dst, pad]).reshape(EROWS, 128)
    xp = jnp.pad(x, ((0, NROWS - N), (0, 0)))
    b1r = b1.reshape(1, D)
    b2r = b2.reshape(1, D)

    t1p = _mm_pad(xp, W1)
    agg1 = _sc_agg(t1p, src2d, dst2d)
    h1p, t2p = _mid(agg1, t1p, b1r, W2)
    agg2 = _sc_agg(t2p, src2d, dst2d)
    h2p = _fin(agg2, t2p, b2r)
    return h1p[:N], h2p[:N]


# speedup vs baseline: 3.2172x; 1.1179x over previous
import functools

import jax
import jax.numpy as jnp
from jax import lax
from jax.experimental import pallas as pl
from jax.experimental.pallas import tpu as pltpu
from jax.experimental.pallas import tpu_sc as plsc

N = 10000
D = 128
DW = 144
NROWS = 10016
E = 320000
EPAD = 327680
EROWS = 2560
SENT = N
NWORK = 32
RPW = EROWS // NWORK
NB = 2
CH = 4
TPT = NROWS // 16

BLK = 2504


def _ones_tail(nrows):
    return jnp.where(
        lax.broadcasted_iota(jnp.int32, (nrows, 16), 1) == 0, 1.0, 0.0
    ).astype(jnp.float32)


def _mm_pad_body(x_ref, w_ref, o_ref):
    t = jnp.dot(x_ref[...], w_ref[...], preferred_element_type=jnp.float32)
    o_ref[...] = jnp.concatenate([t, _ones_tail(BLK)], axis=1)


def _mm_pad(xp, W):
    return pl.pallas_call(
        _mm_pad_body,
        grid=(NROWS // BLK,),
        in_specs=[
            pl.BlockSpec((BLK, D), lambda i: (i, 0)),
            pl.BlockSpec((D, D), lambda i: (0, 0)),
        ],
        out_specs=pl.BlockSpec((BLK, DW), lambda i: (i, 0)),
        out_shape=jax.ShapeDtypeStruct((NROWS, DW), jnp.float32),
    )(xp, W)


def _mid_body(agg_ref, t_ref, b_ref, w_ref, h1_ref, t2_ref):
    s = agg_ref[0] + agg_ref[1]
    deg = s[:, 128:129]
    h = (s[:, :128] + t_ref[:, :128]) / (deg + 1.0) + b_ref[...]
    h1 = jnp.maximum(h, 0.0)
    h1_ref[...] = h1
    t2 = jnp.dot(h1, w_ref[...], preferred_element_type=jnp.float32)
    t2_ref[...] = jnp.concatenate([t2, _ones_tail(BLK)], axis=1)


def _mid(agg, t1p, b1, W2):
    return pl.pallas_call(
        _mid_body,
        grid=(NROWS // BLK,),
        in_specs=[
            pl.BlockSpec((2, BLK, DW), lambda i: (0, i, 0)),
            pl.BlockSpec((BLK, DW), lambda i: (i, 0)),
            pl.BlockSpec((1, D), lambda i: (0, 0)),
            pl.BlockSpec((D, D), lambda i: (0, 0)),
        ],
        out_specs=[
            pl.BlockSpec((BLK, D), lambda i: (i, 0)),
            pl.BlockSpec((BLK, DW), lambda i: (i, 0)),
        ],
        out_shape=[
            jax.ShapeDtypeStruct((NROWS, D), jnp.float32),
            jax.ShapeDtypeStruct((NROWS, DW), jnp.float32),
        ],
    )(agg, t1p, b1, W2)


def _fin_body(agg_ref, t_ref, b_ref, h2_ref):
    s = agg_ref[0] + agg_ref[1]
    deg = s[:, 128:129]
    h2_ref[...] = (s[:, :128] + t_ref[:, :128]) / (deg + 1.0) + b_ref[...]


def _fin(agg, t2p, b2):
    return pl.pallas_call(
        _fin_body,
        grid=(NROWS // BLK,),
        in_specs=[
            pl.BlockSpec((2, BLK, DW), lambda i: (0, i, 0)),
            pl.BlockSpec((BLK, DW), lambda i: (i, 0)),
            pl.BlockSpec((1, D), lambda i: (0, 0)),
        ],
        out_specs=pl.BlockSpec((BLK, D), lambda i: (i, 0)),
        out_shape=jax.ShapeDtypeStruct((NROWS, D), jnp.float32),
    )(agg, t2p, b2)


def _sc_agg_body(t_hbm, src_hbm, dst_hbm, out_hbm, src_v, dst_v, rows_v,
                 acc_sh, *sems):
    gs = sems[:NB]
    ss = sems[NB:]
    cid = lax.axis_index("c")
    sid = lax.axis_index("s")
    wid = cid * 16 + sid

    z16 = jnp.zeros((16,), jnp.float32)
    for r in range(128):
        for c in range(DW // 16):
            rows_v[0, r, 16 * c:16 * (c + 1)] = z16
    for k in range(4):
        pltpu.sync_copy(rows_v.at[0],
                        acc_sh.at[pl.ds(sid * TPT + 128 * k, 128)])
    pltpu.sync_copy(rows_v.at[0, pl.ds(0, TPT - 512)],
                    acc_sh.at[pl.ds(sid * TPT + 512, TPT - 512)])

    wbase = wid * RPW
    pltpu.sync_copy(src_hbm.at[pl.ds(wbase, CH)], src_v.at[0])
    pltpu.sync_copy(dst_hbm.at[pl.ds(wbase, CH)], dst_v.at[0])
    plsc.subcore_barrier()

    g_desc = [
        pltpu.async_copy(t_hbm.at[src_v.at[0, b]], rows_v.at[b], gs[b])
        for b in range(NB)
    ]
    for j in range(RPW):
        b = j % NB
        c = j // CH
        if j % CH == 0 and j + CH < RPW:
            pltpu.sync_copy(src_hbm.at[pl.ds(wbase + j + CH, CH)],
                            src_v.at[(c + 1) % 2])
            pltpu.sync_copy(dst_hbm.at[pl.ds(wbase + j + CH, CH)],
                            dst_v.at[(c + 1) % 2])
        g_desc[b].wait()
        s = pltpu.async_copy(rows_v.at[b],
                             acc_sh.at[dst_v.at[c % 2, j % CH]], ss[b],
                             add=True)
        s.wait()
        if j + NB < RPW:
            jn = j + NB
            g_desc[b] = pltpu.async_copy(
                t_hbm.at[src_v.at[(jn // CH) % 2, jn % CH]], rows_v.at[b],
                gs[b])

    plsc.subcore_barrier()
    pltpu.sync_copy(acc_sh.at[pl.ds(sid * TPT, TPT)],
                    out_hbm.at[cid, pl.ds(sid * TPT, TPT)])


@functools.partial(
    pl.kernel,
    mesh=plsc.VectorSubcoreMesh(core_axis_name="c", subcore_axis_name="s"),
    compiler_params=pltpu.CompilerParams(use_tc_tiling_on_sc=False),
    out_type=jax.ShapeDtypeStruct((2, NROWS, DW), jnp.float32),
    scratch_types=[
        pltpu.VMEM((2, CH, 128), jnp.int32),
        pltpu.VMEM((2, CH, 128), jnp.int32),
        pltpu.VMEM((NB, 128, DW), jnp.float32),
        pltpu.VMEM_SHARED((NROWS, DW), jnp.float32),
    ] + [pltpu.SemaphoreType.DMA] * (2 * NB),
)
def _sc_agg(t_hbm, src_hbm, dst_hbm, out_hbm, src_v, dst_v, rows_v,
            acc_sh, *sems):
    _sc_agg_body(t_hbm, src_hbm, dst_hbm, out_hbm, src_v, dst_v, rows_v,
                 acc_sh, *sems)


def kernel(x, edge_index, W1, b1, W2, b2):
    src = edge_index[0]
    dst = edge_index[1]
    pad = jnp.full((EPAD - E,), SENT, jnp.int32)
    src2d = jnp.concatenate([src, pad]).reshape(EROWS, 128)
    dst2d = jnp.concatenate([dst, pad]).reshape(EROWS, 128)
    xp = jnp.pad(x, ((0, NROWS - N), (0, 0)))
    b1r = b1.reshape(1, D)
    b2r = b2.reshape(1, D)

    t1p = _mm_pad(xp, W1)
    agg1 = _sc_agg(t1p, src2d, dst2d)
    h1p, t2p = _mid(agg1, t1p, b1r, W2)
    agg2 = _sc_agg(t2p, src2d, dst2d)
    h2p = _fin(agg2, t2p, b2r)
    return h1p[:N], h2p[:N]
